# Initial kernel scaffold; baseline (speedup 1.0000x reference)
import functools

import numpy as np
import jax
import jax.numpy as jnp
from jax import lax
from jax.experimental import pallas as pl
from jax.experimental.pallas import tpu as pltpu
from jax.experimental.pallas import tpu_sc as plsc

_NUM_LEVELS = 16
_LOG2_T = 14
_T = 1 << _LOG2_T
_F = 8
_N = 131072
_GROWTH = np.exp((np.log(1024.0) - np.log(16.0)) / (_NUM_LEVELS - 1))
_RES = np.floor(16.0 * _GROWTH ** np.arange(_NUM_LEVELS)).astype(np.float32)

_K2 = np.uint32(2654435761).astype(np.int32)
_K3 = np.int32(805459861)
_MASK = np.int32(_T - 1)

_NC, _NS = 2, 16
_NW = _NC * _NS
_PW = _N // _NW
_P = 256
_NCHUNK = _PW // _P
_G = _P // 16


def _body(xyz, table, resb, out, xyz_v, res_v, w_v, idx_v, rows_v, out_v,
          sem_in, sem_g, sem_out):
    wid = lax.axis_index("s") * _NC + lax.axis_index("c")
    lanes = lax.iota(jnp.int32, 16)

    pltpu.async_copy(resb, res_v, sem_in).wait()

    def chunk_body(ci, carry):
        base = wid * _PW + ci * _P
        pltpu.async_copy(xyz.at[:, pl.ds(base, _P)], xyz_v, sem_in).wait()

        def level_body(l, carry2):
            res_vec = res_v[pl.ds(l * 16, 16)]
            loff = l * _T

            def hash_body(g, carry3):
                o = g * 16
                x = xyz_v[0, pl.ds(o, 16)]
                y = xyz_v[1, pl.ds(o, 16)]
                z = xyz_v[2, pl.ds(o, 16)]
                sx = x * res_vec
                sy = y * res_vec
                sz = z * res_vec
                fx = sx.astype(jnp.int32)
                fy = sy.astype(jnp.int32)
                fz = sz.astype(jnp.int32)
                w_v[0, pl.ds(o, 16)] = sx - fx.astype(jnp.float32)
                w_v[1, pl.ds(o, 16)] = sy - fy.astype(jnp.float32)
                w_v[2, pl.ds(o, 16)] = sz - fz.astype(jnp.float32)
                b0 = fy * _K2
                c0 = fz * _K3
                ax = (fx, fx + 1)
                by = (b0, b0 + _K2)
                cz = (c0, c0 + _K3)
                for zb in range(2):
                    for yb in range(2):
                        for xb in range(2):
                            h = ((ax[xb] ^ by[yb] ^ cz[zb]) & _MASK) + loff
                            slot = xb + 2 * yb + 4 * zb
                            idx_v[pl.ds(slot * _P + o, 16)] = h
                return carry3

            lax.fori_loop(0, _G, hash_body, 0)

            pltpu.async_copy(table.at[idx_v], rows_v, sem_g).wait()

            def interp_body(g, carry3):
                o = g * 16
                wx = w_v[0, pl.ds(o, 16)]
                wy = w_v[1, pl.ds(o, 16)]
                wz = w_v[2, pl.ds(o, 16)]
                r = o + lanes
                rows = [r + (s * _P) for s in range(8)]
                for f in range(_F):
                    col = jnp.full((16,), f, jnp.int32)
                    v = [plsc.load_gather(rows_v, [rows[s], col])
                         for s in range(8)]
                    g00 = v[0] + wx * (v[1] - v[0])
                    g10 = v[2] + wx * (v[3] - v[2])
                    g01 = v[4] + wx * (v[5] - v[4])
                    g11 = v[6] + wx * (v[7] - v[6])
                    h0 = g00 + wy * (g10 - g00)
                    h1 = g01 + wy * (g11 - g01)
                    e = h0 + wz * (h1 - h0)
                    ocol = jnp.broadcast_to(l * _F + f, (16,)).astype(jnp.int32)
                    plsc.store_scatter(out_v, [r, ocol], e)
                return carry3

            lax.fori_loop(0, _G, interp_body, 0)
            return carry2

        lax.fori_loop(0, _NUM_LEVELS, level_body, 0)
        pltpu.async_copy(out_v, out.at[pl.ds(base, _P)], sem_out).wait()
        return carry

    lax.fori_loop(0, _NCHUNK, chunk_body, 0)


_hash_enc = functools.partial(
    pl.kernel,
    out_type=jax.ShapeDtypeStruct((_N, _NUM_LEVELS * _F), jnp.float32),
    mesh=plsc.VectorSubcoreMesh(core_axis_name="c", subcore_axis_name="s"),
    scratch_types=[
        pltpu.VMEM((3, _P), jnp.float32),
        pltpu.VMEM((16 * 16,), jnp.float32),
        pltpu.VMEM((3, _P), jnp.float32),
        pltpu.VMEM((8 * _P,), jnp.int32),
        pltpu.VMEM((8 * _P, _F), jnp.float32),
        pltpu.VMEM((_P, _NUM_LEVELS * _F), jnp.float32),
        pltpu.SemaphoreType.DMA,
        pltpu.SemaphoreType.DMA,
        pltpu.SemaphoreType.DMA,
    ],
)(_body)


def kernel(inp_points, hash_table):
    xyz = inp_points.T
    resb = jnp.asarray(np.repeat(_RES, 16))
    return _hash_enc(xyz, hash_table, resb)


# speedup vs baseline: 112.3784x; 112.3784x over previous
import functools

import numpy as np
import jax
import jax.numpy as jnp
from jax import lax
from jax.experimental import pallas as pl
from jax.experimental.pallas import tpu as pltpu
from jax.experimental.pallas import tpu_sc as plsc

_NUM_LEVELS = 16
_LOG2_T = 14
_T = 1 << _LOG2_T
_F = 8
_N = 131072
_GROWTH = np.exp((np.log(1024.0) - np.log(16.0)) / (_NUM_LEVELS - 1))
_RES = np.floor(16.0 * _GROWTH ** np.arange(_NUM_LEVELS)).astype(np.float32)

_K2 = np.uint32(2654435761).astype(np.int32)
_K3 = np.int32(805459861)
_MASK = np.int32(_T - 1)

_NC, _NS = 2, 16
_NW = _NC * _NS
_PW = _N // _NW
_P = 256
_NCHUNK = _PW // _P
_G = _P // 16


def _body(xyz, table, resb, out, xyz_v, res_v, w_v, idx_v, rows_v, out_v,
          sem_in, sem_g, sem_out):
    wid = lax.axis_index("s") * _NC + lax.axis_index("c")
    lanes = lax.iota(jnp.int32, 16)

    pltpu.async_copy(resb, res_v, sem_in).wait()

    def chunk_body(ci, carry):
        base = wid * jnp.int32(_PW) + ci * jnp.int32(_P)
        pltpu.async_copy(xyz.at[:, pl.ds(base, _P)], xyz_v, sem_in).wait()

        def level_body(l, carry2):
            res_vec = res_v[pl.ds(l * jnp.int32(16), 16)]
            loff = l * jnp.int32(_T)

            def hash_body(g, carry3):
                o = g * jnp.int32(16)
                x = xyz_v[0, pl.ds(o, 16)]
                y = xyz_v[1, pl.ds(o, 16)]
                z = xyz_v[2, pl.ds(o, 16)]
                sx = x * res_vec
                sy = y * res_vec
                sz = z * res_vec
                fx = sx.astype(jnp.int32)
                fy = sy.astype(jnp.int32)
                fz = sz.astype(jnp.int32)
                w_v[0, pl.ds(o, 16)] = sx - fx.astype(jnp.float32)
                w_v[1, pl.ds(o, 16)] = sy - fy.astype(jnp.float32)
                w_v[2, pl.ds(o, 16)] = sz - fz.astype(jnp.float32)
                b0 = fy * _K2
                c0 = fz * _K3
                ax = (fx, fx + jnp.int32(1))
                by = (b0, b0 + _K2)
                cz = (c0, c0 + _K3)
                for zb in range(2):
                    for yb in range(2):
                        for xb in range(2):
                            h = ((ax[xb] ^ by[yb] ^ cz[zb]) & _MASK) + loff
                            slot = xb + 2 * yb + 4 * zb
                            idx_v[pl.ds(jnp.int32(slot * _P) + o, 16)] = h
                return carry3

            lax.fori_loop(jnp.int32(0), jnp.int32(_G), hash_body, jnp.int32(0))

            pltpu.async_copy(table.at[idx_v], rows_v, sem_g).wait()

            def interp_body(g, carry3):
                o = g * jnp.int32(16)
                wx = w_v[0, pl.ds(o, 16)]
                wy = w_v[1, pl.ds(o, 16)]
                wz = w_v[2, pl.ds(o, 16)]
                r = o + lanes
                rows = [r + jnp.int32(s * _P) for s in range(8)]
                for f in range(_F):
                    col = jnp.full((16,), f, jnp.int32)
                    v = [plsc.load_gather(rows_v, [rows[s], col])
                         for s in range(8)]
                    g00 = v[0] + wx * (v[1] - v[0])
                    g10 = v[2] + wx * (v[3] - v[2])
                    g01 = v[4] + wx * (v[5] - v[4])
                    g11 = v[6] + wx * (v[7] - v[6])
                    h0 = g00 + wy * (g10 - g00)
                    h1 = g01 + wy * (g11 - g01)
                    e = h0 + wz * (h1 - h0)
                    ocol = jnp.broadcast_to(l * jnp.int32(_F) + jnp.int32(f), (16,)).astype(jnp.int32)
                    plsc.store_scatter(out_v, [r, ocol], e)
                return carry3

            lax.fori_loop(jnp.int32(0), jnp.int32(_G), interp_body, jnp.int32(0))
            return carry2

        lax.fori_loop(jnp.int32(0), jnp.int32(_NUM_LEVELS), level_body, jnp.int32(0))
        pltpu.async_copy(out_v, out.at[pl.ds(base, _P)], sem_out).wait()
        return carry

    lax.fori_loop(jnp.int32(0), jnp.int32(_NCHUNK), chunk_body, jnp.int32(0))


_hash_enc = functools.partial(
    pl.kernel,
    out_type=jax.ShapeDtypeStruct((_N, _NUM_LEVELS * _F), jnp.float32),
    mesh=plsc.VectorSubcoreMesh(core_axis_name="c", subcore_axis_name="s"),
    scratch_types=[
        pltpu.VMEM((3, _P), jnp.float32),
        pltpu.VMEM((16 * 16,), jnp.float32),
        pltpu.VMEM((3, _P), jnp.float32),
        pltpu.VMEM((8 * _P,), jnp.int32),
        pltpu.VMEM((8 * _P, _F), jnp.float32),
        pltpu.VMEM((_P, _NUM_LEVELS * _F), jnp.float32),
        pltpu.SemaphoreType.DMA,
        pltpu.SemaphoreType.DMA,
        pltpu.SemaphoreType.DMA,
    ],
    compiler_params=pltpu.CompilerParams(
        needs_layout_passes=False, use_tc_tiling_on_sc=False),
)(_body)


def kernel(inp_points, hash_table):
    xyz = inp_points.T
    resb = jnp.asarray(np.repeat(_RES, 16))
    return _hash_enc(xyz, hash_table, resb)


# speedup vs baseline: 195.9719x; 1.7439x over previous
import functools

import numpy as np
import jax
import jax.numpy as jnp
from jax import lax
from jax.experimental import pallas as pl
from jax.experimental.pallas import tpu as pltpu
from jax.experimental.pallas import tpu_sc as plsc

_NUM_LEVELS = 16
_LOG2_T = 14
_T = 1 << _LOG2_T
_F = 8
_N = 131072
_GROWTH = np.exp((np.log(1024.0) - np.log(16.0)) / (_NUM_LEVELS - 1))
_RES = np.floor(16.0 * _GROWTH ** np.arange(_NUM_LEVELS)).astype(np.float32)

_K2 = np.uint32(2654435761).astype(np.int32)
_K3 = np.int32(805459861)
_MASK = np.int32(_T - 1)

_NC, _NS = 2, 16
_NW = _NC * _NS
_PW = _N // _NW
_P = 256
_NCHUNK = _PW // _P
_G = _P // 16


def _body(xyz, table, resb, out, xyz_v, res_v, w0_v, w1_v, idx0_v, idx1_v,
          rows0_v, rows1_v, out_v, sem_in, sem_g0, sem_g1, sem_out):
    wid = lax.axis_index("s") * jnp.int32(_NC) + lax.axis_index("c")
    lanes = lax.iota(jnp.int32, 16)

    pltpu.async_copy(resb, res_v, sem_in).wait()

    def hash_pass(l, idx_v, w_v):
        res_vec = res_v[pl.ds(l * jnp.int32(16), 16)]
        loff = l * jnp.int32(_T)

        def hash_body(g, carry):
            o = g * jnp.int32(16)
            x = xyz_v[0, pl.ds(o, 16)]
            y = xyz_v[1, pl.ds(o, 16)]
            z = xyz_v[2, pl.ds(o, 16)]
            sx = x * res_vec
            sy = y * res_vec
            sz = z * res_vec
            fx = sx.astype(jnp.int32)
            fy = sy.astype(jnp.int32)
            fz = sz.astype(jnp.int32)
            w_v[0, pl.ds(o, 16)] = sx - fx.astype(jnp.float32)
            w_v[1, pl.ds(o, 16)] = sy - fy.astype(jnp.float32)
            w_v[2, pl.ds(o, 16)] = sz - fz.astype(jnp.float32)
            b0 = fy * _K2
            c0 = fz * _K3
            ax = (fx, fx + jnp.int32(1))
            by = (b0, b0 + _K2)
            cz = (c0, c0 + _K3)
            for zb in range(2):
                for yb in range(2):
                    for xb in range(2):
                        h = ((ax[xb] ^ by[yb] ^ cz[zb]) & _MASK) + loff
                        slot = xb + 2 * yb + 4 * zb
                        idx_v[pl.ds(jnp.int32(slot * _P) + o, 16)] = h
            return carry

        lax.fori_loop(jnp.int32(0), jnp.int32(_G), hash_body, jnp.int32(0))

    def interp_pass(l, rows_v, w_v):
        lf = l * jnp.int32(_F)

        def interp_body(g, carry):
            o = g * jnp.int32(16)
            wx = w_v[0, pl.ds(o, 16)]
            wy = w_v[1, pl.ds(o, 16)]
            wz = w_v[2, pl.ds(o, 16)]
            ux = 1.0 - wx
            uy = 1.0 - wy
            uz = 1.0 - wz
            p00 = ux * uy
            p10 = wx * uy
            p01 = ux * wy
            p11 = wx * wy
            w8 = [p00 * uz, p10 * uz, p01 * uz, p11 * uz,
                  p00 * wz, p10 * wz, p01 * wz, p11 * wz]
            r = o + lanes
            rows = [r + jnp.int32(s * _P) for s in range(8)]
            for f in range(_F):
                col = jnp.full((16,), f, jnp.int32)
                acc = None
                for s in range(8):
                    v = plsc.load_gather(rows_v, [rows[s], col])
                    t = v * w8[s]
                    acc = t if acc is None else acc + t
                ocol = jnp.broadcast_to(lf + jnp.int32(f), (16,))
                plsc.store_scatter(out_v, [r, ocol], acc)
            return carry

        lax.fori_loop(jnp.int32(0), jnp.int32(_G), interp_body, jnp.int32(0))

    def chunk_body(ci, carry):
        base = wid * jnp.int32(_PW) + ci * jnp.int32(_P)
        pltpu.async_copy(xyz.at[:, pl.ds(base, _P)], xyz_v, sem_in).wait()

        hash_pass(jnp.int32(0), idx0_v, w0_v)
        pltpu.async_copy(table.at[idx0_v], rows0_v, sem_g0)

        def dbl_body(k, carry2):
            l0 = k * jnp.int32(2)
            l1 = l0 + jnp.int32(1)
            hash_pass(l1, idx1_v, w1_v)
            pltpu.async_copy(table.at[idx1_v], rows1_v, sem_g1)
            pltpu.make_async_copy(table.at[idx0_v], rows0_v, sem_g0).wait()
            interp_pass(l0, rows0_v, w0_v)

            @pl.when(k < jnp.int32(_NUM_LEVELS // 2 - 1))
            def _prefetch_next():
                hash_pass(l0 + jnp.int32(2), idx0_v, w0_v)
                pltpu.async_copy(table.at[idx0_v], rows0_v, sem_g0)

            pltpu.make_async_copy(table.at[idx1_v], rows1_v, sem_g1).wait()
            interp_pass(l1, rows1_v, w1_v)
            return carry2

        lax.fori_loop(jnp.int32(0), jnp.int32(_NUM_LEVELS // 2), dbl_body,
                      jnp.int32(0))
        pltpu.async_copy(out_v, out.at[pl.ds(base, _P)], sem_out).wait()
        return carry

    lax.fori_loop(jnp.int32(0), jnp.int32(_NCHUNK), chunk_body, jnp.int32(0))


_hash_enc = functools.partial(
    pl.kernel,
    out_type=jax.ShapeDtypeStruct((_N, _NUM_LEVELS * _F), jnp.float32),
    mesh=plsc.VectorSubcoreMesh(core_axis_name="c", subcore_axis_name="s"),
    scratch_types=[
        pltpu.VMEM((3, _P), jnp.float32),
        pltpu.VMEM((16 * 16,), jnp.float32),
        pltpu.VMEM((3, _P), jnp.float32),
        pltpu.VMEM((3, _P), jnp.float32),
        pltpu.VMEM((8 * _P,), jnp.int32),
        pltpu.VMEM((8 * _P,), jnp.int32),
        pltpu.VMEM((8 * _P, _F), jnp.float32),
        pltpu.VMEM((8 * _P, _F), jnp.float32),
        pltpu.VMEM((_P, _NUM_LEVELS * _F), jnp.float32),
        pltpu.SemaphoreType.DMA,
        pltpu.SemaphoreType.DMA,
        pltpu.SemaphoreType.DMA,
        pltpu.SemaphoreType.DMA,
    ],
    compiler_params=pltpu.CompilerParams(
        needs_layout_passes=False, use_tc_tiling_on_sc=False),
)(_body)


def kernel(inp_points, hash_table):
    xyz = inp_points.T
    resb = jnp.asarray(np.repeat(_RES, 16))
    return _hash_enc(xyz, hash_table, resb)


# speedup vs baseline: 268.5046x; 1.3701x over previous
import functools

import numpy as np
import jax
import jax.numpy as jnp
from jax import lax
from jax.experimental import pallas as pl
from jax.experimental.pallas import tpu as pltpu
from jax.experimental.pallas import tpu_sc as plsc

_NUM_LEVELS = 16
_LOG2_T = 14
_T = 1 << _LOG2_T
_F = 8
_N = 131072
_GROWTH = np.exp((np.log(1024.0) - np.log(16.0)) / (_NUM_LEVELS - 1))
_RES = np.floor(16.0 * _GROWTH ** np.arange(_NUM_LEVELS)).astype(np.float32)

_K2 = np.uint32(2654435761).astype(np.int32)
_K3 = np.int32(805459861)
_MASK = np.int32(_T - 1)

_NC, _NS = 2, 16
_NW = _NC * _NS
_PW = _N // _NW
_P = 256
_NCHUNK = _PW // _P
_G = _P // 16


def _body(xyz, table, resb, out, xyz_v, res_v, w0_v, w1_v, idx0_v, idx1_v,
          rows0_v, rows1_v, out_v, sem_in, sem_g0, sem_g1, sem_out):
    wid = lax.axis_index("s") * jnp.int32(_NC) + lax.axis_index("c")
    lanes = lax.iota(jnp.int32, 16)

    pltpu.async_copy(resb, res_v, sem_in).wait()

    def hash_pass(l, idx_v, w_v):
        res_vec = res_v[pl.ds(l * jnp.int32(16), 16)]
        loff = l * jnp.int32(_T)

        def hash_body(g, carry):
            o = g * jnp.int32(16)
            x = xyz_v[0, pl.ds(o, 16)]
            y = xyz_v[1, pl.ds(o, 16)]
            z = xyz_v[2, pl.ds(o, 16)]
            sx = x * res_vec
            sy = y * res_vec
            sz = z * res_vec
            fx = sx.astype(jnp.int32)
            fy = sy.astype(jnp.int32)
            fz = sz.astype(jnp.int32)
            w_v[0, pl.ds(o, 16)] = sx - fx.astype(jnp.float32)
            w_v[1, pl.ds(o, 16)] = sy - fy.astype(jnp.float32)
            w_v[2, pl.ds(o, 16)] = sz - fz.astype(jnp.float32)
            b0 = fy * _K2
            c0 = fz * _K3
            ax = (fx, fx + jnp.int32(1))
            by = (b0, b0 + _K2)
            cz = (c0, c0 + _K3)
            for zb in range(2):
                for yb in range(2):
                    for xb in range(2):
                        h = ((ax[xb] ^ by[yb] ^ cz[zb]) & _MASK) + loff
                        slot = xb + 2 * yb + 4 * zb
                        idx_v[pl.ds(jnp.int32(slot * _P) + o, 16)] = h
            return carry

        lax.fori_loop(jnp.int32(0), jnp.int32(_G), hash_body, jnp.int32(0))

    def interp_pass(l, rows_v, w_v):
        lf = l * jnp.int32(_F)
        cols = [jnp.full((16,), f, jnp.int32) for f in range(_F)]

        @plsc.parallel_loop(jnp.int32(0), jnp.int32(_G), jnp.int32(1),
                            unroll=2)
        def interp_body(g):
            o = g * jnp.int32(16)
            wx = w_v[0, pl.ds(o, 16)]
            wy = w_v[1, pl.ds(o, 16)]
            wz = w_v[2, pl.ds(o, 16)]
            ux = 1.0 - wx
            uy = 1.0 - wy
            uz = 1.0 - wz
            p00 = ux * uy
            p10 = wx * uy
            p01 = ux * wy
            p11 = wx * wy
            w8 = [p00 * uz, p10 * uz, p01 * uz, p11 * uz,
                  p00 * wz, p10 * wz, p01 * wz, p11 * wz]
            r = o + lanes
            rows = [r + jnp.int32(s * _P) for s in range(8)]
            accs = [None] * _F
            for s in range(8):
                for f in range(_F):
                    v = plsc.load_gather(rows_v, [rows[s], cols[f]])
                    t = v * w8[s]
                    accs[f] = t if accs[f] is None else accs[f] + t
            for f in range(_F):
                ocol = jnp.broadcast_to(lf + jnp.int32(f), (16,))
                plsc.store_scatter(out_v, [r, ocol], accs[f])

    def chunk_body(ci, carry):
        base = wid * jnp.int32(_PW) + ci * jnp.int32(_P)
        pltpu.async_copy(xyz.at[:, pl.ds(base, _P)], xyz_v, sem_in).wait()

        hash_pass(jnp.int32(0), idx0_v, w0_v)
        pltpu.async_copy(table.at[idx0_v], rows0_v, sem_g0)

        def dbl_body(k, carry2):
            l0 = k * jnp.int32(2)
            l1 = l0 + jnp.int32(1)
            hash_pass(l1, idx1_v, w1_v)
            pltpu.async_copy(table.at[idx1_v], rows1_v, sem_g1)
            pltpu.make_async_copy(table.at[idx0_v], rows0_v, sem_g0).wait()
            interp_pass(l0, rows0_v, w0_v)

            @pl.when(k < jnp.int32(_NUM_LEVELS // 2 - 1))
            def _prefetch_next():
                hash_pass(l0 + jnp.int32(2), idx0_v, w0_v)
                pltpu.async_copy(table.at[idx0_v], rows0_v, sem_g0)

            pltpu.make_async_copy(table.at[idx1_v], rows1_v, sem_g1).wait()
            interp_pass(l1, rows1_v, w1_v)
            return carry2

        lax.fori_loop(jnp.int32(0), jnp.int32(_NUM_LEVELS // 2), dbl_body,
                      jnp.int32(0))
        pltpu.async_copy(out_v, out.at[pl.ds(base, _P)], sem_out).wait()
        return carry

    lax.fori_loop(jnp.int32(0), jnp.int32(_NCHUNK), chunk_body, jnp.int32(0))


_hash_enc = functools.partial(
    pl.kernel,
    out_type=jax.ShapeDtypeStruct((_N, _NUM_LEVELS * _F), jnp.float32),
    mesh=plsc.VectorSubcoreMesh(core_axis_name="c", subcore_axis_name="s"),
    scratch_types=[
        pltpu.VMEM((3, _P), jnp.float32),
        pltpu.VMEM((16 * 16,), jnp.float32),
        pltpu.VMEM((3, _P), jnp.float32),
        pltpu.VMEM((3, _P), jnp.float32),
        pltpu.VMEM((8 * _P,), jnp.int32),
        pltpu.VMEM((8 * _P,), jnp.int32),
        pltpu.VMEM((8 * _P, _F), jnp.float32),
        pltpu.VMEM((8 * _P, _F), jnp.float32),
        pltpu.VMEM((_P, _NUM_LEVELS * _F), jnp.float32),
        pltpu.SemaphoreType.DMA,
        pltpu.SemaphoreType.DMA,
        pltpu.SemaphoreType.DMA,
        pltpu.SemaphoreType.DMA,
    ],
    compiler_params=pltpu.CompilerParams(
        needs_layout_passes=False, use_tc_tiling_on_sc=False),
)(_body)


def kernel(inp_points, hash_table):
    xyz = inp_points.T
    resb = jnp.asarray(np.repeat(_RES, 16))
    return _hash_enc(xyz, hash_table, resb)


# speedup vs baseline: 291.5830x; 1.0860x over previous
import functools

import numpy as np
import jax
import jax.numpy as jnp
from jax import lax
from jax.experimental import pallas as pl
from jax.experimental.pallas import tpu as pltpu
from jax.experimental.pallas import tpu_sc as plsc

_NUM_LEVELS = 16
_LOG2_T = 14
_T = 1 << _LOG2_T
_F = 8
_N = 131072
_GROWTH = np.exp((np.log(1024.0) - np.log(16.0)) / (_NUM_LEVELS - 1))
_RES = np.floor(16.0 * _GROWTH ** np.arange(_NUM_LEVELS)).astype(np.float32)

_K2 = np.uint32(2654435761).astype(np.int32)
_K3 = np.int32(805459861)
_MASK = np.int32(_T - 1)

_NC, _NS = 2, 16
_NW = _NC * _NS
_PW = _N // _NW
_P = 256
_NCHUNK = _PW // _P
_G = _P // 16


def _body(xyz, table, resb, out, xyz_v, res_v, w0_v, w1_v, idx0_v, idx1_v,
          rows0_v, rows1_v, out_v, sem_in, sem_g0, sem_g0b, sem_g1,
          sem_g1b, sem_out):
    wid = lax.axis_index("s") * jnp.int32(_NC) + lax.axis_index("c")
    lanes = lax.iota(jnp.int32, 16)

    pltpu.async_copy(resb, res_v, sem_in).wait()

    def hash_pass(l, idx_v, w_v):
        res_vec = res_v[pl.ds(l * jnp.int32(16), 16)]
        loff = l * jnp.int32(_T)

        def hash_body(g, carry):
            o = g * jnp.int32(16)
            x = xyz_v[0, pl.ds(o, 16)]
            y = xyz_v[1, pl.ds(o, 16)]
            z = xyz_v[2, pl.ds(o, 16)]
            sx = x * res_vec
            sy = y * res_vec
            sz = z * res_vec
            fx = sx.astype(jnp.int32)
            fy = sy.astype(jnp.int32)
            fz = sz.astype(jnp.int32)
            w_v[0, pl.ds(o, 16)] = sx - fx.astype(jnp.float32)
            w_v[1, pl.ds(o, 16)] = sy - fy.astype(jnp.float32)
            w_v[2, pl.ds(o, 16)] = sz - fz.astype(jnp.float32)
            b0 = fy * _K2
            c0 = fz * _K3
            ax = (fx, fx + jnp.int32(1))
            by = (b0, b0 + _K2)
            cz = (c0, c0 + _K3)
            for zb in range(2):
                for yb in range(2):
                    for xb in range(2):
                        h = ((ax[xb] ^ by[yb] ^ cz[zb]) & _MASK) + loff
                        slot = xb + 2 * yb + 4 * zb
                        idx_v[pl.ds(jnp.int32(slot * _P) + o, 16)] = h
            return carry

        lax.fori_loop(jnp.int32(0), jnp.int32(_G), hash_body, jnp.int32(0))

    def interp_pass(l, rows_v, w_v):
        lf = l * jnp.int32(_F)
        cols = [jnp.full((16,), f, jnp.int32) for f in range(_F)]

        @plsc.parallel_loop(jnp.int32(0), jnp.int32(_G), jnp.int32(1),
                            unroll=2)
        def interp_body(g):
            o = g * jnp.int32(16)
            wx = w_v[0, pl.ds(o, 16)]
            wy = w_v[1, pl.ds(o, 16)]
            wz = w_v[2, pl.ds(o, 16)]
            ux = 1.0 - wx
            uy = 1.0 - wy
            uz = 1.0 - wz
            p00 = ux * uy
            p10 = wx * uy
            p01 = ux * wy
            p11 = wx * wy
            w8 = [p00 * uz, p10 * uz, p01 * uz, p11 * uz,
                  p00 * wz, p10 * wz, p01 * wz, p11 * wz]
            r = o + lanes
            rows = [r + jnp.int32(s * _P) for s in range(8)]
            accs = [None] * _F
            for s in range(8):
                for f in range(_F):
                    v = plsc.load_gather(rows_v, [rows[s], cols[f]])
                    t = v * w8[s]
                    accs[f] = t if accs[f] is None else accs[f] + t
            for f in range(_F):
                ocol = jnp.broadcast_to(lf + jnp.int32(f), (16,))
                plsc.store_scatter(out_v, [r, ocol], accs[f])

    _H = 4 * _P

    def start_gather(idx_v, rows_v, sa, sb):
        pltpu.async_copy(table.at[idx_v.at[pl.ds(0, _H)]],
                         rows_v.at[pl.ds(0, _H)], sa)
        pltpu.async_copy(table.at[idx_v.at[pl.ds(_H, _H)]],
                         rows_v.at[pl.ds(_H, _H)], sb)

    def wait_gather(idx_v, rows_v, sa, sb):
        pltpu.make_async_copy(table.at[idx_v.at[pl.ds(0, _H)]],
                              rows_v.at[pl.ds(0, _H)], sa).wait()
        pltpu.make_async_copy(table.at[idx_v.at[pl.ds(_H, _H)]],
                              rows_v.at[pl.ds(_H, _H)], sb).wait()

    def chunk_body(ci, carry):
        base = wid * jnp.int32(_PW) + ci * jnp.int32(_P)
        pltpu.async_copy(xyz.at[:, pl.ds(base, _P)], xyz_v, sem_in).wait()

        hash_pass(jnp.int32(0), idx0_v, w0_v)
        start_gather(idx0_v, rows0_v, sem_g0, sem_g0b)

        def dbl_body(k, carry2):
            l0 = k * jnp.int32(2)
            l1 = l0 + jnp.int32(1)
            hash_pass(l1, idx1_v, w1_v)
            start_gather(idx1_v, rows1_v, sem_g1, sem_g1b)
            wait_gather(idx0_v, rows0_v, sem_g0, sem_g0b)
            interp_pass(l0, rows0_v, w0_v)

            @pl.when(k < jnp.int32(_NUM_LEVELS // 2 - 1))
            def _prefetch_next():
                hash_pass(l0 + jnp.int32(2), idx0_v, w0_v)
                start_gather(idx0_v, rows0_v, sem_g0, sem_g0b)

            wait_gather(idx1_v, rows1_v, sem_g1, sem_g1b)
            interp_pass(l1, rows1_v, w1_v)
            return carry2

        lax.fori_loop(jnp.int32(0), jnp.int32(_NUM_LEVELS // 2), dbl_body,
                      jnp.int32(0))
        pltpu.async_copy(out_v, out.at[pl.ds(base, _P)], sem_out).wait()
        return carry

    lax.fori_loop(jnp.int32(0), jnp.int32(_NCHUNK), chunk_body, jnp.int32(0))


_hash_enc = functools.partial(
    pl.kernel,
    out_type=jax.ShapeDtypeStruct((_N, _NUM_LEVELS * _F), jnp.float32),
    mesh=plsc.VectorSubcoreMesh(core_axis_name="c", subcore_axis_name="s"),
    scratch_types=[
        pltpu.VMEM((3, _P), jnp.float32),
        pltpu.VMEM((16 * 16,), jnp.float32),
        pltpu.VMEM((3, _P), jnp.float32),
        pltpu.VMEM((3, _P), jnp.float32),
        pltpu.VMEM((8 * _P,), jnp.int32),
        pltpu.VMEM((8 * _P,), jnp.int32),
        pltpu.VMEM((8 * _P, _F), jnp.float32),
        pltpu.VMEM((8 * _P, _F), jnp.float32),
        pltpu.VMEM((_P, _NUM_LEVELS * _F), jnp.float32),
        pltpu.SemaphoreType.DMA,
        pltpu.SemaphoreType.DMA,
        pltpu.SemaphoreType.DMA,
        pltpu.SemaphoreType.DMA,
        pltpu.SemaphoreType.DMA,
        pltpu.SemaphoreType.DMA,
    ],
    compiler_params=pltpu.CompilerParams(
        needs_layout_passes=False, use_tc_tiling_on_sc=False),
)(_body)


def kernel(inp_points, hash_table):
    xyz = inp_points.T
    resb = jnp.asarray(np.repeat(_RES, 16))
    return _hash_enc(xyz, hash_table, resb)
